# Initial kernel scaffold; baseline (speedup 1.0000x reference)
import jax
import jax.numpy as jnp
from jax.experimental import pallas as pl


def kernel(x, w1, shift1, w2, shift2, wfc, bfc):
    raise NotImplementedError("write your pallas kernel here")


# speedup vs baseline: 67.5838x; 67.5838x over previous
import jax
import jax.numpy as jnp
from jax.experimental import pallas as pl
from jax.experimental.pallas import tpu as pltpu

LANE = 128
C1 = 16
C2 = 32
NCLS = 27
W1PAD = 18
K1 = 5 * 32
N1 = 2 * W1PAD * C1
K2 = 5 * W1PAD * C1
N2 = 2 * 7 * C2


def _build_t1(w1):
    W1 = w1[:, :C1].reshape(5, 5, C1)
    wi = jnp.arange(32)
    wp = jnp.arange(W1PAD)
    dw = jnp.arange(2)
    wo = 2 * (wp[None, :] - 2) + dw[:, None]
    kj = wi[None, None, :] - wo[:, :, None]
    valid = (kj >= 0) & (kj < 5) & (wp[None, :, None] >= 2) & (wp[None, :, None] <= 15)
    g = W1[:, jnp.clip(kj, 0, 4), :]
    g = g * valid[None, :, :, :, None].astype(g.dtype)
    t1 = jnp.transpose(g, (0, 3, 1, 2, 4)).reshape(K1, N1)
    return t1


def _build_t2(w2):
    W2 = w2[:, :C2].reshape(5, 5, C1, C2)
    wp = jnp.arange(W1PAD)
    pw = jnp.arange(7)
    dw = jnp.arange(2)
    wo = 2 * pw[None, :] + dw[:, None]
    kj = wp[None, None, :] - wo[:, :, None]
    valid = (kj >= 0) & (kj < 5)
    g = W2[:, jnp.clip(kj, 0, 4), :, :]
    g = g * valid[None, :, :, :, None, None].astype(g.dtype)
    t2 = jnp.transpose(g, (0, 3, 4, 1, 2, 5)).reshape(K2, N2)
    return t2


def _fused_kernel(xp_ref, t1_ref, s1_ref, t2_ref, s2_ref, wfc_ref, bfc_ref,
                  o_ref):
    bt = xp_ref.shape[0]
    f32 = jnp.float32
    xv = xp_ref[...]
    t1 = t1_ref[...]
    s1 = s1_ref[...]

    y1 = []
    for ph in range(14):
        accs = []
        for dh in range(2):
            h = 2 * ph + dh
            xrow = jnp.concatenate([xv[:, h + i:h + i + 1, :] for i in range(5)],
                                   axis=2).reshape(bt, K1)
            accs.append(jnp.dot(xrow, t1, preferred_element_type=f32))
        m = jnp.maximum(jnp.maximum(accs[0] + s1, 0.0),
                        jnp.maximum(accs[1] + s1, 0.0))
        y = jnp.maximum(m[:, :N1 // 2], m[:, N1 // 2:])
        y1.append(y.astype(jnp.bfloat16))

    zrow = jnp.zeros((bt, W1PAD * C1), jnp.bfloat16)
    y1pad = [zrow, zrow] + y1 + [zrow, zrow]

    t2 = t2_ref[...]
    s2 = s2_ref[...]
    acc = jnp.zeros((bt, LANE), f32)
    for p in range(7):
        ms = []
        for dh in range(2):
            h2 = 2 * p + dh
            r = jnp.concatenate([y1pad[h2 + ki] for ki in range(5)], axis=1)
            a = jnp.dot(r, t2, preferred_element_type=f32)
            ms.append(jnp.maximum(a + s2, 0.0))
        m = jnp.maximum(ms[0], ms[1])
        y2 = jnp.maximum(m[:, :N2 // 2], m[:, N2 // 2:]).astype(jnp.bfloat16)
        acc = acc + jnp.dot(y2, wfc_ref[224 * p:224 * (p + 1), :],
                            preferred_element_type=f32)
    o_ref[...] = acc + bfc_ref[...]


def kernel(x, w1, shift1, w2, shift2, wfc, bfc):
    B = x.shape[0]
    BT = 128
    xp = jnp.pad(x.reshape(B, 28, 28), ((0, 0), (2, 2), (2, 2))).astype(jnp.bfloat16)
    t1 = _build_t1(w1)
    t2 = _build_t2(w2)
    wp = jnp.arange(W1PAD)
    wvalid = ((wp >= 2) & (wp <= 15)).astype(jnp.float32)
    s1t = (shift1[0, :C1][None, :] * wvalid[:, None]).reshape(1, W1PAD * C1)
    s1t = jnp.concatenate([s1t, s1t], axis=1)
    s2t = jnp.tile(shift2[:, :C2], (1, 14)).reshape(1, N2)

    out = pl.pallas_call(
        _fused_kernel,
        grid=(B // BT,),
        out_shape=jax.ShapeDtypeStruct((B, LANE), jnp.float32),
        in_specs=[
            pl.BlockSpec((BT, 32, 32), lambda b: (b, 0, 0)),
            pl.BlockSpec((K1, N1), lambda b: (0, 0)),
            pl.BlockSpec((1, N1), lambda b: (0, 0)),
            pl.BlockSpec((K2, N2), lambda b: (0, 0)),
            pl.BlockSpec((1, N2), lambda b: (0, 0)),
            pl.BlockSpec((7 * 224, LANE), lambda b: (0, 0)),
            pl.BlockSpec((1, LANE), lambda b: (0, 0)),
        ],
        out_specs=pl.BlockSpec((BT, LANE), lambda b: (b, 0)),
        compiler_params=pltpu.CompilerParams(dimension_semantics=("parallel",)),
    )(xp, t1, s1t, t2, s2t, wfc, bfc)
    return out[:, :NCLS]


# speedup vs baseline: 100.6180x; 1.4888x over previous
import jax
import jax.numpy as jnp
from jax.experimental import pallas as pl
from jax.experimental.pallas import tpu as pltpu

LANE = 128
C1 = 16
C2 = 32
NCLS = 27
W1PAD = 24
K1 = 5 * 32
N1 = 2 * W1PAD * C1
K2 = 5 * W1PAD * C1
N2 = 2 * 7 * C2


def _build_t1(w1):
    W1 = w1[:, :C1].reshape(5, 5, C1)
    wi = jnp.arange(32)
    wp = jnp.arange(W1PAD)
    dw = jnp.arange(2)
    wo = 2 * (wp[None, :] - 2) + dw[:, None]
    kj = wi[None, None, :] - wo[:, :, None]
    valid = (kj >= 0) & (kj < 5) & (wp[None, :, None] >= 2) & (wp[None, :, None] <= 15)
    g = W1[:, jnp.clip(kj, 0, 4), :]
    g = g * valid[None, :, :, :, None].astype(g.dtype)
    t1 = jnp.transpose(g, (0, 3, 1, 2, 4)).reshape(K1, N1)
    return t1


def _build_t2(w2):
    W2 = w2[:, :C2].reshape(5, 5, C1, C2)
    wp = jnp.arange(W1PAD)
    pw = jnp.arange(7)
    dw = jnp.arange(2)
    wo = 2 * pw[None, :] + dw[:, None]
    kj = wp[None, None, :] - wo[:, :, None]
    valid = (kj >= 0) & (kj < 5)
    g = W2[:, jnp.clip(kj, 0, 4), :, :]
    g = g * valid[None, :, :, :, None, None].astype(g.dtype)
    t2 = jnp.transpose(g, (0, 3, 4, 1, 2, 5)).reshape(K2, N2)
    return t2


def _fused_kernel(xp_ref, t1_ref, s1_ref, t2_ref, s2_ref, wfc_ref, bfc_ref,
                  o_ref):
    bt = xp_ref.shape[0]
    f32 = jnp.float32
    xv = xp_ref[...]
    t1 = t1_ref[...]
    s1 = s1_ref[...]

    y1 = []
    for ph in range(14):
        accs = []
        for dh in range(2):
            h = 2 * ph + dh
            xrow = xv[:, 32 * h:32 * h + K1]
            accs.append(jnp.dot(xrow, t1, preferred_element_type=f32))
        m = jnp.maximum(jnp.maximum(accs[0] + s1, 0.0),
                        jnp.maximum(accs[1] + s1, 0.0))
        y = jnp.maximum(m[:, :N1 // 2], m[:, N1 // 2:])
        y1.append(y.astype(jnp.bfloat16))

    zrow = jnp.zeros((bt, W1PAD * C1), jnp.bfloat16)
    y1pad = [zrow, zrow] + y1 + [zrow, zrow]

    t2 = t2_ref[...]
    s2 = s2_ref[...]
    acc = jnp.zeros((bt, LANE), f32)
    for p in range(7):
        ms = []
        for dh in range(2):
            h2 = 2 * p + dh
            r = jnp.concatenate([y1pad[h2 + ki] for ki in range(5)], axis=1)
            a = jnp.dot(r, t2, preferred_element_type=f32)
            ms.append(jnp.maximum(a + s2, 0.0))
        m = jnp.maximum(ms[0], ms[1])
        y2 = jnp.maximum(m[:, :N2 // 2], m[:, N2 // 2:]).astype(jnp.bfloat16)
        acc = acc + jnp.dot(y2, wfc_ref[224 * p:224 * (p + 1), :],
                            preferred_element_type=f32)
    o_ref[...] = acc + bfc_ref[...]


def kernel(x, w1, shift1, w2, shift2, wfc, bfc):
    B = x.shape[0]
    BT = 128
    xp = jnp.pad(x.reshape(B, 28, 28), ((0, 0), (2, 2), (2, 2)))
    xp = xp.astype(jnp.bfloat16).reshape(B, 1024)
    t1 = _build_t1(w1)
    t2 = _build_t2(w2)
    wp = jnp.arange(W1PAD)
    wvalid = ((wp >= 2) & (wp <= 15)).astype(jnp.float32)
    s1t = (shift1[0, :C1][None, :] * wvalid[:, None]).reshape(1, W1PAD * C1)
    s1t = jnp.concatenate([s1t, s1t], axis=1)
    s2t = jnp.tile(shift2[:, :C2], (1, 14)).reshape(1, N2)

    out = pl.pallas_call(
        _fused_kernel,
        grid=(B // BT,),
        out_shape=jax.ShapeDtypeStruct((B, LANE), jnp.float32),
        in_specs=[
            pl.BlockSpec((BT, 1024), lambda b: (b, 0)),
            pl.BlockSpec((K1, N1), lambda b: (0, 0)),
            pl.BlockSpec((1, N1), lambda b: (0, 0)),
            pl.BlockSpec((K2, N2), lambda b: (0, 0)),
            pl.BlockSpec((1, N2), lambda b: (0, 0)),
            pl.BlockSpec((7 * 224, LANE), lambda b: (0, 0)),
            pl.BlockSpec((1, LANE), lambda b: (0, 0)),
        ],
        out_specs=pl.BlockSpec((BT, LANE), lambda b: (b, 0)),
        compiler_params=pltpu.CompilerParams(dimension_semantics=("parallel",)),
    )(xp, t1, s1t, t2, s2t, wfc, bfc)
    return out[:, :NCLS]


# speedup vs baseline: 150.7942x; 1.4987x over previous
import jax
import jax.numpy as jnp
from jax.experimental import pallas as pl
from jax.experimental.pallas import tpu as pltpu

LANE = 128
C1 = 16
C2 = 32
NCLS = 27
WS = 16
K1 = 5 * 32
N1 = 2 * WS * C1
PIECE = WS * C1
K2 = 5 * PIECE
N2 = 2 * 7 * C2


def _build_t1(w1):
    W1 = w1[:, :C1].reshape(5, 5, C1)
    wi = jnp.arange(32)
    pw = jnp.arange(WS)
    dw = jnp.arange(2)
    wo = 2 * pw[None, :] + dw[:, None]
    kj = wi[None, None, :] - wo[:, :, None]
    valid = (kj >= 0) & (kj < 5) & (pw[None, :, None] <= 13)
    g = W1[:, jnp.clip(kj, 0, 4), :]
    g = g * valid[None, :, :, :, None].astype(g.dtype)
    t1 = jnp.transpose(g, (0, 3, 1, 2, 4)).reshape(K1, N1)
    return t1


def _build_t2(w2):
    W2 = w2[:, :C2].reshape(5, 5, C1, C2)
    ws = jnp.arange(WS)
    pw = jnp.arange(7)
    dw = jnp.arange(2)
    wo = 2 * pw[None, :] + dw[:, None]
    kj = ws[None, None, :] - wo[:, :, None] + 2
    valid = (kj >= 0) & (kj < 5) & (ws[None, None, :] <= 13)
    g = W2[:, jnp.clip(kj, 0, 4), :, :]
    g = g * valid[None, :, :, :, None, None].astype(g.dtype)
    t2 = jnp.transpose(g, (0, 3, 4, 1, 2, 5)).reshape(K2, N2)
    return t2


def _fused_kernel(xp_ref, t1_ref, s1_ref, t2_ref, s2_ref, wfc_ref, bfc_ref,
                  o_ref):
    bt = xp_ref.shape[0]
    f32 = jnp.float32
    xv = xp_ref[...]
    t1 = t1_ref[...]
    s1 = s1_ref[...]

    y1 = []
    for ph in range(14):
        accs = []
        for dh in range(2):
            h = 2 * ph + dh
            xrow = xv[:, 32 * h:32 * h + K1]
            accs.append(jnp.dot(xrow, t1, preferred_element_type=f32))
        m = jnp.maximum(jnp.maximum(accs[0] + s1, 0.0),
                        jnp.maximum(accs[1] + s1, 0.0))
        y = jnp.maximum(m[:, :N1 // 2], m[:, N1 // 2:])
        y1.append(y.astype(jnp.bfloat16))

    s2 = s2_ref[...]
    acc = jnp.zeros((bt, LANE), f32)
    for p in range(7):
        ms = []
        for dh in range(2):
            h2 = 2 * p + dh
            lo = max(0, h2 - 2)
            hi = min(13, h2 + 2)
            r = jnp.concatenate([y1[h] for h in range(lo, hi + 1)], axis=1) \
                if hi > lo else y1[lo]
            tb = t2_ref[PIECE * (lo - h2 + 2):PIECE * (hi - h2 + 3), :]
            a = jnp.dot(r, tb, preferred_element_type=f32)
            ms.append(jnp.maximum(a + s2, 0.0))
        m = jnp.maximum(ms[0], ms[1])
        y2 = jnp.maximum(m[:, :N2 // 2], m[:, N2 // 2:]).astype(jnp.bfloat16)
        acc = acc + jnp.dot(y2, wfc_ref[224 * p:224 * (p + 1), :],
                            preferred_element_type=f32)
    o_ref[...] = acc + bfc_ref[...]


def kernel(x, w1, shift1, w2, shift2, wfc, bfc):
    B = x.shape[0]
    BT = 128
    xp = jnp.pad(x.reshape(B, 28, 28), ((0, 0), (2, 2), (2, 2)))
    xp = xp.astype(jnp.bfloat16).reshape(B, 1024)
    t1 = _build_t1(w1)
    t2 = _build_t2(w2)
    pw = jnp.arange(WS)
    wvalid = (pw <= 13).astype(jnp.float32)
    s1t = (shift1[0, :C1][None, :] * wvalid[:, None]).reshape(1, PIECE)
    s1t = jnp.concatenate([s1t, s1t], axis=1)
    s2t = jnp.tile(shift2[:, :C2], (1, 14)).reshape(1, N2)

    out = pl.pallas_call(
        _fused_kernel,
        grid=(B // BT,),
        out_shape=jax.ShapeDtypeStruct((B, LANE), jnp.float32),
        in_specs=[
            pl.BlockSpec((BT, 1024), lambda b: (b, 0)),
            pl.BlockSpec((K1, N1), lambda b: (0, 0)),
            pl.BlockSpec((1, N1), lambda b: (0, 0)),
            pl.BlockSpec((K2, N2), lambda b: (0, 0)),
            pl.BlockSpec((1, N2), lambda b: (0, 0)),
            pl.BlockSpec((7 * 224, LANE), lambda b: (0, 0)),
            pl.BlockSpec((1, LANE), lambda b: (0, 0)),
        ],
        out_specs=pl.BlockSpec((BT, LANE), lambda b: (b, 0)),
        compiler_params=pltpu.CompilerParams(dimension_semantics=("parallel",)),
    )(xp, t1, s1t, t2, s2t, wfc, bfc)
    return out[:, :NCLS]


# speedup vs baseline: 175.2502x; 1.1622x over previous
import jax
import jax.numpy as jnp
from jax.experimental import pallas as pl
from jax.experimental.pallas import tpu as pltpu

LANE = 128
C1 = 16
C2 = 32
NCLS = 27
WS = 16
K1 = 5 * 32
N1 = 2 * WS * C1
PIECE = WS * C1
K2 = 5 * PIECE
N2 = 2 * 7 * C2


def _build_t1(w1):
    W1 = w1[:, :C1].reshape(5, 5, C1)
    wi = jnp.arange(32)
    pw = jnp.arange(WS)
    dw = jnp.arange(2)
    wo = 2 * pw[None, :] + dw[:, None]
    kj = wi[None, None, :] - wo[:, :, None]
    valid = (kj >= 0) & (kj < 5) & (pw[None, :, None] <= 13)
    g = W1[:, jnp.clip(kj, 0, 4), :]
    g = g * valid[None, :, :, :, None].astype(g.dtype)
    t1 = jnp.transpose(g, (0, 3, 1, 2, 4)).reshape(K1, N1)
    return t1


def _build_t2(w2):
    W2 = w2[:, :C2].reshape(5, 5, C1, C2)
    ws = jnp.arange(WS)
    pw = jnp.arange(7)
    dw = jnp.arange(2)
    wo = 2 * pw[None, :] + dw[:, None]
    kj = ws[None, None, :] - wo[:, :, None] + 2
    valid = (kj >= 0) & (kj < 5) & (ws[None, None, :] <= 13)
    g = W2[:, jnp.clip(kj, 0, 4), :, :]
    g = g * valid[None, :, :, :, None, None].astype(g.dtype)
    t2 = jnp.transpose(g, (0, 3, 4, 1, 2, 5)).reshape(K2, N2)
    return t2


def _fused_kernel(xp_ref, t1_ref, s1_ref, t2_ref, s2_ref, wfc_ref, bfc_ref,
                  o_ref):
    bt = xp_ref.shape[0]
    f32 = jnp.float32
    xv = xp_ref[...]
    t1 = t1_ref[...]
    s1 = s1_ref[...]

    y1 = []
    for ph in range(14):
        accs = []
        for dh in range(2):
            h = 2 * ph + dh
            xrow = xv[:, 32 * h:32 * h + K1]
            accs.append(jnp.dot(xrow, t1, preferred_element_type=f32))
        m = jnp.maximum(jnp.maximum(accs[0] + s1, 0.0),
                        jnp.maximum(accs[1] + s1, 0.0))
        y = jnp.maximum(m[:, :N1 // 2], m[:, N1 // 2:])
        y1.append(y.astype(jnp.bfloat16))

    s2 = s2_ref[...]
    acc = jnp.zeros((bt, LANE), f32)
    for p in range(7):
        ms = []
        for dh in range(2):
            h2 = 2 * p + dh
            lo = max(0, h2 - 2)
            hi = min(13, h2 + 2)
            r = jnp.concatenate([y1[h] for h in range(lo, hi + 1)], axis=1) \
                if hi > lo else y1[lo]
            tb = t2_ref[PIECE * (lo - h2 + 2):PIECE * (hi - h2 + 3), :]
            a = jnp.dot(r, tb, preferred_element_type=f32)
            ms.append(jnp.maximum(a + s2, 0.0))
        m = jnp.maximum(ms[0], ms[1])
        y2 = jnp.maximum(m[:, :N2 // 2], m[:, N2 // 2:]).astype(jnp.bfloat16)
        acc = acc + jnp.dot(y2, wfc_ref[224 * p:224 * (p + 1), :],
                            preferred_element_type=f32)
    o_ref[...] = acc + bfc_ref[...]


def kernel(x, w1, shift1, w2, shift2, wfc, bfc):
    B = x.shape[0]
    BT = 256
    xp = jnp.pad(x.reshape(B, 28, 28), ((0, 0), (2, 2), (2, 2)))
    xp = xp.astype(jnp.bfloat16).reshape(B, 1024)
    t1 = _build_t1(w1)
    t2 = _build_t2(w2)
    pw = jnp.arange(WS)
    wvalid = (pw <= 13).astype(jnp.float32)
    s1t = (shift1[0, :C1][None, :] * wvalid[:, None]).reshape(1, PIECE)
    s1t = jnp.concatenate([s1t, s1t], axis=1)
    s2t = jnp.tile(shift2[:, :C2], (1, 14)).reshape(1, N2)

    out = pl.pallas_call(
        _fused_kernel,
        grid=(B // BT,),
        out_shape=jax.ShapeDtypeStruct((B, LANE), jnp.float32),
        in_specs=[
            pl.BlockSpec((BT, 1024), lambda b: (b, 0)),
            pl.BlockSpec((K1, N1), lambda b: (0, 0)),
            pl.BlockSpec((1, N1), lambda b: (0, 0)),
            pl.BlockSpec((K2, N2), lambda b: (0, 0)),
            pl.BlockSpec((1, N2), lambda b: (0, 0)),
            pl.BlockSpec((7 * 224, LANE), lambda b: (0, 0)),
            pl.BlockSpec((1, LANE), lambda b: (0, 0)),
        ],
        out_specs=pl.BlockSpec((BT, LANE), lambda b: (b, 0)),
        compiler_params=pltpu.CompilerParams(dimension_semantics=("parallel",)),
    )(xp, t1, s1t, t2, s2t, wfc, bfc)
    return out[:, :NCLS]


# speedup vs baseline: 188.4614x; 1.0754x over previous
import jax
import jax.numpy as jnp
from jax.experimental import pallas as pl
from jax.experimental.pallas import tpu as pltpu

LANE = 128
C1 = 16
C2 = 32
NCLS = 27
WS = 16
K1 = 5 * 32
N1 = 2 * WS * C1
PIECE = WS * C1
K2 = 5 * PIECE
N2 = 2 * 7 * C2


def _build_t1(w1):
    W1 = w1[:, :C1].reshape(5, 5, C1)
    wi = jnp.arange(32)
    pw = jnp.arange(WS)
    dw = jnp.arange(2)
    wo = 2 * pw[None, :] + dw[:, None]
    kj = wi[None, None, :] - wo[:, :, None]
    valid = (kj >= 0) & (kj < 5) & (pw[None, :, None] <= 13)
    g = W1[:, jnp.clip(kj, 0, 4), :]
    g = g * valid[None, :, :, :, None].astype(g.dtype)
    t1 = jnp.transpose(g, (0, 3, 1, 2, 4)).reshape(K1, N1)
    return t1


def _build_t2(w2):
    W2 = w2[:, :C2].reshape(5, 5, C1, C2)
    ws = jnp.arange(WS)
    pw = jnp.arange(7)
    dw = jnp.arange(2)
    wo = 2 * pw[None, :] + dw[:, None]
    kj = ws[None, None, :] - wo[:, :, None] + 2
    valid = (kj >= 0) & (kj < 5) & (ws[None, None, :] <= 13)
    g = W2[:, jnp.clip(kj, 0, 4), :, :]
    g = g * valid[None, :, :, :, None, None].astype(g.dtype)
    t2 = jnp.transpose(g, (0, 3, 4, 1, 2, 5)).reshape(K2, N2)
    return t2


def _fused_kernel(xp_ref, t1_ref, s1_ref, t2_ref, s2_ref, wfc_ref, bfc_ref,
                  o_ref):
    bt = xp_ref.shape[0]
    f32 = jnp.float32
    xv = xp_ref[...]
    t1 = t1_ref[...]
    s1 = s1_ref[...]

    y1 = []
    for ph in range(14):
        accs = []
        for dh in range(2):
            h = 2 * ph + dh
            xrow = xv[:, 32 * h:32 * h + K1]
            accs.append(jnp.dot(xrow, t1, preferred_element_type=f32))
        m = jnp.maximum(jnp.maximum(accs[0] + s1, 0.0),
                        jnp.maximum(accs[1] + s1, 0.0))
        y = jnp.maximum(m[:, :N1 // 2], m[:, N1 // 2:])
        y1.append(y.astype(jnp.bfloat16))

    s2 = s2_ref[...]
    acc = jnp.zeros((bt, LANE), f32)
    for p in range(7):
        ms = []
        for dh in range(2):
            h2 = 2 * p + dh
            lo = max(0, h2 - 2)
            hi = min(13, h2 + 2)
            r = jnp.concatenate([y1[h] for h in range(lo, hi + 1)], axis=1) \
                if hi > lo else y1[lo]
            tb = t2_ref[PIECE * (lo - h2 + 2):PIECE * (hi - h2 + 3), :]
            a = jnp.dot(r, tb, preferred_element_type=f32)
            ms.append(jnp.maximum(a + s2, 0.0))
        m = jnp.maximum(ms[0], ms[1])
        y2 = jnp.maximum(m[:, :N2 // 2], m[:, N2 // 2:]).astype(jnp.bfloat16)
        acc = acc + jnp.dot(y2, wfc_ref[224 * p:224 * (p + 1), :],
                            preferred_element_type=f32)
    o_ref[...] = acc + bfc_ref[...]


def kernel(x, w1, shift1, w2, shift2, wfc, bfc):
    B = x.shape[0]
    BT = 512
    xp = jnp.pad(x.reshape(B, 28, 28), ((0, 0), (2, 2), (2, 2)))
    xp = xp.astype(jnp.bfloat16).reshape(B, 1024)
    t1 = _build_t1(w1)
    t2 = _build_t2(w2)
    pw = jnp.arange(WS)
    wvalid = (pw <= 13).astype(jnp.float32)
    s1t = (shift1[0, :C1][None, :] * wvalid[:, None]).reshape(1, PIECE)
    s1t = jnp.concatenate([s1t, s1t], axis=1)
    s2t = jnp.tile(shift2[:, :C2], (1, 14)).reshape(1, N2)

    out = pl.pallas_call(
        _fused_kernel,
        grid=(B // BT,),
        out_shape=jax.ShapeDtypeStruct((B, LANE), jnp.float32),
        in_specs=[
            pl.BlockSpec((BT, 1024), lambda b: (b, 0)),
            pl.BlockSpec((K1, N1), lambda b: (0, 0)),
            pl.BlockSpec((1, N1), lambda b: (0, 0)),
            pl.BlockSpec((K2, N2), lambda b: (0, 0)),
            pl.BlockSpec((1, N2), lambda b: (0, 0)),
            pl.BlockSpec((7 * 224, LANE), lambda b: (0, 0)),
            pl.BlockSpec((1, LANE), lambda b: (0, 0)),
        ],
        out_specs=pl.BlockSpec((BT, LANE), lambda b: (b, 0)),
        compiler_params=pltpu.CompilerParams(dimension_semantics=("parallel",)),
    )(xp, t1, s1t, t2, s2t, wfc, bfc)
    return out[:, :NCLS]
